# baseline (speedup 1.0000x reference)
import functools

import jax
import jax.numpy as jnp
from jax import lax
from jax.experimental import pallas as pl
from jax.experimental.pallas import tpu as pltpu
from jax.experimental.pallas import tpu_sc as plsc

N = 100000
E = 1600000
F_IN = 4
H = 32
HH = H // 2
G = 256
OUT = 2

NC = 2
NS = 16

BN = 4000
NBLK = N // BN
PBN = BN * H // 128
PN = N * H // 128
NPR = 128 // H

EB = 800
E_PER_TILE = E // NS
N_ECHUNK = E_PER_TILE // EB
CH = 800
NCHUNK = N // CH
CH_PER_TILE = (NCHUNK + NS - 1) // NS


def _sc_seg_body(z_hbm, srcA_hbm, srcB_hbm, dst_hbm, out_hbm, agg_s,
                 sv0, sv1, sv2, dv0, dv1, rv0, rv1,
                 isrc0, isrc1, isrc2, idst0, idst1,
                 gsem0, gsem1, ssem0, ssem1):
    c = lax.axis_index("c")
    s = lax.axis_index("s")

    def zero_rows(i, carry):
        rv0[i, :] = jnp.zeros((16,), jnp.float32)
        return carry

    lax.fori_loop(0, CH, zero_rows, 0)
    for k in range(CH_PER_TILE):
        cid = s * CH_PER_TILE + k

        @pl.when(cid < NCHUNK)
        def _():
            pltpu.sync_copy(
                rv0.at[pl.ds(0, CH), :],
                agg_s.at[pl.ds(cid * CH, CH), :],
            )
    plsc.subcore_barrier()

    ebase = s * E_PER_TILE
    srcs, isrc = (sv0, sv1, sv2), (isrc0, isrc1, isrc2)
    dsts, idst = (dv0, dv1), (idst0, idst1)
    rows, gsem = (rv0, rv1), (gsem0, gsem1)
    ssem = (ssem0, ssem1)

    def eoff(i):
        return pl.multiple_of(ebase + i * EB, 8)

    def edge_pipeline(ci):
        src_hbm = srcA_hbm if ci == 0 else srcB_hbm

        def src_start(i, sl):
            pltpu.async_copy(src_hbm.at[pl.ds(eoff(i), EB)], srcs[sl],
                             isrc[sl])

        def src_wait(sl):
            pltpu.make_async_copy(src_hbm.at[pl.ds(0, EB)], srcs[sl],
                                  isrc[sl]).wait()

        def dst_start(i, sl):
            pltpu.async_copy(dst_hbm.at[pl.ds(eoff(i), EB)], dsts[sl],
                             idst[sl])

        def dst_wait(sl):
            pltpu.make_async_copy(dst_hbm.at[pl.ds(0, EB)], dsts[sl],
                                  idst[sl]).wait()

        def gather_start(ss_, sr):
            pltpu.async_copy(z_hbm.at[srcs[ss_]], rows[sr], gsem[sr])

        def gather_wait(ss_, sr):
            pltpu.make_async_copy(z_hbm.at[srcs[ss_]], rows[sr],
                                  gsem[sr]).wait()

        def scat_start(sr, sd):
            pltpu.make_async_copy(rows[sr], agg_s.at[dsts[sd]],
                                  ssem[sd]).start(add=True)

        def scat_wait(sr, sd):
            pltpu.make_async_copy(rows[sr], agg_s.at[dsts[sd]],
                                  ssem[sd]).wait()

        pltpu.sync_copy(src_hbm.at[pl.ds(eoff(0), EB)], srcs[0])
        src_start(1, 1)
        src_start(2, 2)
        dst_start(0, 0)
        gather_start(0, 0)

        def body(j, jm, src_ahead=True, skip_scat_wait=False):
            src_wait(jm % 3)
            if not skip_scat_wait:
                scat_wait(jm % 2, jm % 2)
            gather_start(jm % 3, jm % 2)
            dst_start(j, jm % 2)
            dst_wait((jm - 1) % 2)
            gather_wait((jm - 1) % 3, (jm - 1) % 2)
            scat_start((jm - 1) % 2, (jm - 1) % 2)
            if src_ahead:
                src_start(j + 2, (jm + 2) % 3)

        body(1, 1, skip_scat_wait=True)

        def steady(t, carry):
            j0 = 2 + 6 * t
            for k in range(6):
                body(j0 + k, 2 + k)
            return carry

        lax.fori_loop(0, (N_ECHUNK - 5) // 6, steady, 0)
        body(N_ECHUNK - 3, N_ECHUNK - 3)
        body(N_ECHUNK - 2, N_ECHUNK - 2, src_ahead=False)
        body(N_ECHUNK - 1, N_ECHUNK - 1, src_ahead=False)
        last = N_ECHUNK - 1
        dst_wait(last % 2)
        gather_wait(last % 3, last % 2)
        scat_start(last % 2, last % 2)
        scat_wait((last - 1) % 2, (last - 1) % 2)
        scat_wait(last % 2, last % 2)

    for ci in range(NC):
        @pl.when(c == ci)
        def _(ci=ci):
            edge_pipeline(ci)

    plsc.subcore_barrier()

    for ci in range(NC):
        @pl.when(c == ci)
        def _(ci=ci):
            for k in range(CH_PER_TILE):
                cid = s * CH_PER_TILE + k

                @pl.when(cid < NCHUNK)
                def _():
                    pltpu.sync_copy(
                        agg_s.at[pl.ds(cid * CH, CH), :],
                        out_hbm.at[pl.ds(cid * CH, CH), ci, :],
                    )


@functools.cache
def _get_sc_seg_sum():
    return pl.kernel(
        _sc_seg_body,
        out_type=jax.ShapeDtypeStruct((N, NC, HH), jnp.float32),
        mesh=plsc.VectorSubcoreMesh(core_axis_name="c", subcore_axis_name="s",
                                    num_cores=NC, num_subcores=NS),
        compiler_params=pltpu.CompilerParams(use_tc_tiling_on_sc=False),
        scratch_types=(
            [pltpu.VMEM_SHARED((N, HH), jnp.float32)]
            + [pltpu.VMEM((EB,), jnp.int32)] * 5
            + [pltpu.VMEM((EB, HH), jnp.float32)] * 2
            + [pltpu.SemaphoreType.DMA] * 9
        ),
    )


def _tc0_body(x_ref, wr_ref, wo_ref, z_ref, r_ref):
    x = x_ref[...]
    z_ref[...] = jnp.dot(x, wr_ref[...], preferred_element_type=jnp.float32)
    r_ref[...] = jnp.dot(x, wo_ref[...], preferred_element_type=jnp.float32)


def _tc_layer_body(s_ref, r_ref, b_ref, wr_ref, wo_ref, z_ref, rn_ref):
    x = jnp.maximum(s_ref[...] + b_ref[...] + r_ref[...], 0.0)
    z_ref[...] = jnp.dot(x, wr_ref[...], preferred_element_type=jnp.float32)
    rn_ref[...] = jnp.dot(x, wo_ref[...], preferred_element_type=jnp.float32)


def _tc_pool_body(s_ref, r_ref, b_ref, batch_ref, sums_ref, cnt_ref):
    i = pl.program_id(0)
    x = jnp.maximum(s_ref[...] + b_ref[...] + r_ref[...], 0.0)
    ids = batch_ref[0, 0, :]
    onehot = (ids[:, None] == lax.broadcasted_iota(jnp.int32, (1, G), 1))
    onehot = onehot.astype(jnp.float32)
    ps = jnp.zeros((G, H), jnp.float32)
    for j in range(NPR):
        oh_j = onehot[j * PBN:(j + 1) * PBN, :]
        x_j = x[:, j * H:(j + 1) * H]
        ps = ps + lax.dot_general(oh_j, x_j, (((0,), (0,)), ((), ())),
                                  preferred_element_type=jnp.float32)
    cnt = jnp.sum(onehot, axis=0)[None, :]

    @pl.when(i == 0)
    def _():
        sums_ref[...] = jnp.zeros_like(sums_ref)
        cnt_ref[...] = jnp.zeros_like(cnt_ref)

    sums_ref[...] += ps
    cnt_ref[...] += cnt


def _tc_mlp_body(sums_ref, cnt_ref, w5, b5, w6, b6, w7, b7, w8, b8, wf, bf,
                 out_ref):
    cnt = jnp.maximum(cnt_ref[0, :], 1.0)[:, None]
    p = sums_ref[...] / cnt
    h = jnp.maximum(jnp.dot(p, w5[...], preferred_element_type=jnp.float32)
                    + b5[...], 0.0)
    h = jnp.maximum(jnp.dot(h, w6[...], preferred_element_type=jnp.float32)
                    + b6[...], 0.0)
    h = jnp.maximum(jnp.dot(h, w7[...], preferred_element_type=jnp.float32)
                    + b7[...], 0.0)
    h = jnp.maximum(jnp.dot(h, w8[...], preferred_element_type=jnp.float32)
                    + b8[...], 0.0)
    out_ref[...] = jax.nn.sigmoid(
        jnp.dot(h, wf[...], preferred_element_type=jnp.float32) + bf[...])


def _full_spec(shape):
    return pl.BlockSpec(shape, lambda *args: tuple(0 for _ in shape))


_pk_spec = pl.BlockSpec((PBN, 128), lambda i: (i, 0))
_pk_shape = jax.ShapeDtypeStruct((PN, 128), jnp.float32)

_tc0 = pl.pallas_call(
    _tc0_body,
    grid=(NBLK,),
    in_specs=[_pk_spec, _full_spec((128, 128)), _full_spec((128, 128))],
    out_specs=[_pk_spec, _pk_spec],
    out_shape=[_pk_shape, _pk_shape],
)

_tc_layer = pl.pallas_call(
    _tc_layer_body,
    grid=(NBLK,),
    in_specs=[
        _pk_spec, _pk_spec, _full_spec((1, 128)),
        _full_spec((128, 128)), _full_spec((128, 128)),
    ],
    out_specs=[_pk_spec, _pk_spec],
    out_shape=[_pk_shape, _pk_shape],
)

_tc_pool = pl.pallas_call(
    _tc_pool_body,
    grid=(NBLK,),
    in_specs=[
        _pk_spec, _pk_spec, _full_spec((1, 128)),
        pl.BlockSpec((1, 1, BN), lambda i: (i, 0, 0)),
    ],
    out_specs=[
        pl.BlockSpec((G, H), lambda i: (0, 0)),
        pl.BlockSpec((1, G), lambda i: (0, 0)),
    ],
    out_shape=[
        jax.ShapeDtypeStruct((G, H), jnp.float32),
        jax.ShapeDtypeStruct((1, G), jnp.float32),
    ],
)

_tc_mlp = pl.pallas_call(
    _tc_mlp_body,
    in_specs=[
        _full_spec((G, H)),
        _full_spec((1, G)),
        _full_spec((H, H)), _full_spec((1, H)),
        _full_spec((H, H)), _full_spec((1, H)),
        _full_spec((H, H)), _full_spec((1, H)),
        _full_spec((H, H)), _full_spec((1, H)),
        _full_spec((H, OUT)), _full_spec((1, OUT)),
    ],
    out_specs=pl.BlockSpec((G, OUT), lambda: (0, 0)),
    out_shape=jax.ShapeDtypeStruct((G, OUT), jnp.float32),
)


def _kron4(w):
    return jnp.kron(jnp.eye(NPR, dtype=jnp.float32), w)


@jax.jit
def kernel(node_attr, edge_index, batch,
           Wrel0, brel0, Wroot0,
           Wrel1, brel1, Wroot1,
           Wrel2, brel2, Wroot2,
           W5, b5, W6, b6, W7, b7, W8, b8, Wf, bf):
    src = edge_index[0]
    dst = edge_index[1]
    src2 = src * 2
    src2p1 = src2 + 1
    x0_pk = jnp.pad(node_attr, ((0, 0), (0, H - F_IN))).reshape(PN, 128)
    batch_tp = (batch.reshape(NBLK, BN // NPR, NPR)
                .transpose(0, 2, 1).reshape(NBLK, 1, BN))
    w0r = _kron4(jnp.pad(Wrel0, ((0, H - F_IN), (0, 0))))
    w0o = _kron4(jnp.pad(Wroot0, ((0, H - F_IN), (0, 0))))
    w1r, w1o = _kron4(Wrel1), _kron4(Wroot1)
    w2r, w2o = _kron4(Wrel2), _kron4(Wroot2)
    b0 = jnp.tile(brel0.reshape(1, H), (1, NPR))
    b1 = jnp.tile(brel1.reshape(1, H), (1, NPR))
    b2 = jnp.tile(brel2.reshape(1, H), (1, NPR))

    sc = _get_sc_seg_sum()

    z0, r0 = _tc0(x0_pk, w0r, w0o)
    s0 = sc(z0.reshape(NC * N, HH), src2, src2p1, dst).reshape(PN, 128)
    z1, r1 = _tc_layer(s0, r0, b0, w1r, w1o)
    s1 = sc(z1.reshape(NC * N, HH), src2, src2p1, dst).reshape(PN, 128)
    z2, r2 = _tc_layer(s1, r1, b1, w2r, w2o)
    s2 = sc(z2.reshape(NC * N, HH), src2, src2p1, dst).reshape(PN, 128)
    sums, cnts = _tc_pool(s2, r2, b2, batch_tp)
    out = _tc_mlp(sums, cnts,
                  W5, b5.reshape(1, H), W6, b6.reshape(1, H),
                  W7, b7.reshape(1, H), W8, b8.reshape(1, H),
                  Wf, bf.reshape(1, OUT))
    return out


# speedup vs baseline: 1.0082x; 1.0082x over previous
import functools

import jax
import jax.numpy as jnp
from jax import lax
from jax.experimental import pallas as pl
from jax.experimental.pallas import tpu as pltpu
from jax.experimental.pallas import tpu_sc as plsc

N = 100000
E = 1600000
F_IN = 4
H = 32
HH = H // 2
G = 256
OUT = 2

NC = 2
NS = 16

BN = 4000
NBLK = N // BN
PBN = BN * H // 128
PN = N * H // 128
NPR = 128 // H

EB = 800
E_PER_TILE = E // NS
N_ECHUNK = E_PER_TILE // EB
CH = 800
NCHUNK = N // CH
CH_PER_TILE = (NCHUNK + NS - 1) // NS


def _sc_seg_body(z_hbm, srcA_hbm, srcB_hbm, dst_hbm, out_hbm, agg_s,
                 sv0, sv1, sv2, dv0, dv1, rv0, rv1,
                 isrc0, isrc1, isrc2, idst0, idst1,
                 gsem0, gsem1, ssem0, ssem1):
    c = lax.axis_index("c")
    s = lax.axis_index("s")

    def zero_rows(i, carry):
        rv0[i, :] = jnp.zeros((16,), jnp.float32)
        return carry

    lax.fori_loop(0, CH, zero_rows, 0)
    for k in range(CH_PER_TILE):
        cid = s * CH_PER_TILE + k

        @pl.when(cid < NCHUNK)
        def _():
            pltpu.sync_copy(
                rv0.at[pl.ds(0, CH), :],
                agg_s.at[pl.ds(cid * CH, CH), :],
            )
    plsc.subcore_barrier()

    ebase = s * E_PER_TILE
    srcs, isrc = (sv0, sv1, sv2), (isrc0, isrc1, isrc2)
    dsts, idst = (dv0, dv1), (idst0, idst1)
    rows, gsem = (rv0, rv1), (gsem0, gsem1)
    ssem = (ssem0, ssem1)

    def eoff(i):
        return pl.multiple_of(ebase + i * EB, 8)

    def edge_pipeline(ci):
        src_hbm = srcA_hbm if ci == 0 else srcB_hbm

        def src_start(i, sl):
            pltpu.async_copy(src_hbm.at[pl.ds(eoff(i), EB)], srcs[sl],
                             isrc[sl])

        def src_wait(sl):
            pltpu.make_async_copy(src_hbm.at[pl.ds(0, EB)], srcs[sl],
                                  isrc[sl]).wait()

        def dst_start(i, sl):
            pltpu.async_copy(dst_hbm.at[pl.ds(eoff(i), EB)], dsts[sl],
                             idst[sl])

        def dst_wait(sl):
            pltpu.make_async_copy(dst_hbm.at[pl.ds(0, EB)], dsts[sl],
                                  idst[sl]).wait()

        def gather_start(ss_, sr):
            pltpu.async_copy(z_hbm.at[srcs[ss_]], rows[sr], gsem[sr])

        def gather_wait(ss_, sr):
            pltpu.make_async_copy(z_hbm.at[srcs[ss_]], rows[sr],
                                  gsem[sr]).wait()

        def scat_start(sr, sd):
            pltpu.make_async_copy(rows[sr], agg_s.at[dsts[sd]],
                                  ssem[sd]).start(add=True)

        def scat_wait(sr, sd):
            pltpu.make_async_copy(rows[sr], agg_s.at[dsts[sd]],
                                  ssem[sd]).wait()

        pltpu.sync_copy(src_hbm.at[pl.ds(eoff(0), EB)], srcs[0])
        dst_start(0, 0)
        gather_start(0, 0)
        src_start(1, 1)

        def body(j, jm, last=False, skip_scat_wait=False):
            src_wait(jm % 3)
            if not skip_scat_wait:
                scat_wait(jm % 2, jm % 2)
            gather_start(jm % 3, jm % 2)
            if not last:
                src_start(j + 1, (jm + 1) % 3)
            dst_wait((jm - 1) % 2)
            gather_wait((jm - 1) % 3, (jm - 1) % 2)
            scat_start((jm - 1) % 2, (jm - 1) % 2)
            dst_start(j, jm % 2)

        body(1, 1, skip_scat_wait=True)

        def steady(t, carry):
            j0 = 2 + 6 * t
            for k in range(6):
                body(j0 + k, 2 + k)
            return carry

        lax.fori_loop(0, (N_ECHUNK - 5) // 6, steady, 0)
        for j in (N_ECHUNK - 3, N_ECHUNK - 2):
            body(j, j)
        body(N_ECHUNK - 1, N_ECHUNK - 1, last=True)
        last = N_ECHUNK - 1
        dst_wait(last % 2)
        gather_wait(last % 3, last % 2)
        scat_start(last % 2, last % 2)
        scat_wait((last - 1) % 2, (last - 1) % 2)
        scat_wait(last % 2, last % 2)

    for ci in range(NC):
        @pl.when(c == ci)
        def _(ci=ci):
            edge_pipeline(ci)

    plsc.subcore_barrier()

    for ci in range(NC):
        @pl.when(c == ci)
        def _(ci=ci):
            for k in range(CH_PER_TILE):
                cid = s * CH_PER_TILE + k

                @pl.when(cid < NCHUNK)
                def _():
                    pltpu.sync_copy(
                        agg_s.at[pl.ds(cid * CH, CH), :],
                        out_hbm.at[pl.ds(cid * CH, CH), ci, :],
                    )


@functools.cache
def _get_sc_seg_sum():
    return pl.kernel(
        _sc_seg_body,
        out_type=jax.ShapeDtypeStruct((N, NC, HH), jnp.float32),
        mesh=plsc.VectorSubcoreMesh(core_axis_name="c", subcore_axis_name="s",
                                    num_cores=NC, num_subcores=NS),
        compiler_params=pltpu.CompilerParams(use_tc_tiling_on_sc=False),
        scratch_types=(
            [pltpu.VMEM_SHARED((N, HH), jnp.float32)]
            + [pltpu.VMEM((EB,), jnp.int32)] * 5
            + [pltpu.VMEM((EB, HH), jnp.float32)] * 2
            + [pltpu.SemaphoreType.DMA] * 9
        ),
    )


def _tc0_body(x_ref, wr_ref, wo_ref, z_ref, r_ref):
    x = x_ref[...]
    z_ref[...] = jnp.dot(x, wr_ref[...], preferred_element_type=jnp.float32)
    r_ref[...] = jnp.dot(x, wo_ref[...], preferred_element_type=jnp.float32)


def _tc_layer_body(s_ref, r_ref, b_ref, wr_ref, wo_ref, z_ref, rn_ref):
    x = jnp.maximum(s_ref[...] + b_ref[...] + r_ref[...], 0.0)
    z_ref[...] = jnp.dot(x, wr_ref[...], preferred_element_type=jnp.float32)
    rn_ref[...] = jnp.dot(x, wo_ref[...], preferred_element_type=jnp.float32)


def _tc_pool_body(s_ref, r_ref, b_ref, batch_ref, sums_ref, cnt_ref):
    i = pl.program_id(0)
    x = jnp.maximum(s_ref[...] + b_ref[...] + r_ref[...], 0.0)
    ids = batch_ref[0, 0, :]
    onehot = (ids[:, None] == lax.broadcasted_iota(jnp.int32, (1, G), 1))
    onehot = onehot.astype(jnp.float32)
    ps = jnp.zeros((G, H), jnp.float32)
    for j in range(NPR):
        oh_j = onehot[j * PBN:(j + 1) * PBN, :]
        x_j = x[:, j * H:(j + 1) * H]
        ps = ps + lax.dot_general(oh_j, x_j, (((0,), (0,)), ((), ())),
                                  preferred_element_type=jnp.float32)
    cnt = jnp.sum(onehot, axis=0)[None, :]

    @pl.when(i == 0)
    def _():
        sums_ref[...] = jnp.zeros_like(sums_ref)
        cnt_ref[...] = jnp.zeros_like(cnt_ref)

    sums_ref[...] += ps
    cnt_ref[...] += cnt


def _tc_mlp_body(sums_ref, cnt_ref, w5, b5, w6, b6, w7, b7, w8, b8, wf, bf,
                 out_ref):
    cnt = jnp.maximum(cnt_ref[0, :], 1.0)[:, None]
    p = sums_ref[...] / cnt
    h = jnp.maximum(jnp.dot(p, w5[...], preferred_element_type=jnp.float32)
                    + b5[...], 0.0)
    h = jnp.maximum(jnp.dot(h, w6[...], preferred_element_type=jnp.float32)
                    + b6[...], 0.0)
    h = jnp.maximum(jnp.dot(h, w7[...], preferred_element_type=jnp.float32)
                    + b7[...], 0.0)
    h = jnp.maximum(jnp.dot(h, w8[...], preferred_element_type=jnp.float32)
                    + b8[...], 0.0)
    out_ref[...] = jax.nn.sigmoid(
        jnp.dot(h, wf[...], preferred_element_type=jnp.float32) + bf[...])


def _full_spec(shape):
    return pl.BlockSpec(shape, lambda *args: tuple(0 for _ in shape))


_pk_spec = pl.BlockSpec((PBN, 128), lambda i: (i, 0))
_pk_shape = jax.ShapeDtypeStruct((PN, 128), jnp.float32)

_tc0 = pl.pallas_call(
    _tc0_body,
    grid=(NBLK,),
    in_specs=[_pk_spec, _full_spec((128, 128)), _full_spec((128, 128))],
    out_specs=[_pk_spec, _pk_spec],
    out_shape=[_pk_shape, _pk_shape],
)

_tc_layer = pl.pallas_call(
    _tc_layer_body,
    grid=(NBLK,),
    in_specs=[
        _pk_spec, _pk_spec, _full_spec((1, 128)),
        _full_spec((128, 128)), _full_spec((128, 128)),
    ],
    out_specs=[_pk_spec, _pk_spec],
    out_shape=[_pk_shape, _pk_shape],
)

_tc_pool = pl.pallas_call(
    _tc_pool_body,
    grid=(NBLK,),
    in_specs=[
        _pk_spec, _pk_spec, _full_spec((1, 128)),
        pl.BlockSpec((1, 1, BN), lambda i: (i, 0, 0)),
    ],
    out_specs=[
        pl.BlockSpec((G, H), lambda i: (0, 0)),
        pl.BlockSpec((1, G), lambda i: (0, 0)),
    ],
    out_shape=[
        jax.ShapeDtypeStruct((G, H), jnp.float32),
        jax.ShapeDtypeStruct((1, G), jnp.float32),
    ],
)

_tc_mlp = pl.pallas_call(
    _tc_mlp_body,
    in_specs=[
        _full_spec((G, H)),
        _full_spec((1, G)),
        _full_spec((H, H)), _full_spec((1, H)),
        _full_spec((H, H)), _full_spec((1, H)),
        _full_spec((H, H)), _full_spec((1, H)),
        _full_spec((H, H)), _full_spec((1, H)),
        _full_spec((H, OUT)), _full_spec((1, OUT)),
    ],
    out_specs=pl.BlockSpec((G, OUT), lambda: (0, 0)),
    out_shape=jax.ShapeDtypeStruct((G, OUT), jnp.float32),
)


def _kron4(w):
    return jnp.kron(jnp.eye(NPR, dtype=jnp.float32), w)


@jax.jit
def kernel(node_attr, edge_index, batch,
           Wrel0, brel0, Wroot0,
           Wrel1, brel1, Wroot1,
           Wrel2, brel2, Wroot2,
           W5, b5, W6, b6, W7, b7, W8, b8, Wf, bf):
    src = edge_index[0]
    dst = edge_index[1]
    src2 = src * 2
    src2p1 = src2 + 1
    x0_pk = jnp.pad(node_attr, ((0, 0), (0, H - F_IN))).reshape(PN, 128)
    batch_tp = (batch.reshape(NBLK, BN // NPR, NPR)
                .transpose(0, 2, 1).reshape(NBLK, 1, BN))
    w0r = _kron4(jnp.pad(Wrel0, ((0, H - F_IN), (0, 0))))
    w0o = _kron4(jnp.pad(Wroot0, ((0, H - F_IN), (0, 0))))
    w1r, w1o = _kron4(Wrel1), _kron4(Wroot1)
    w2r, w2o = _kron4(Wrel2), _kron4(Wroot2)
    b0 = jnp.tile(brel0.reshape(1, H), (1, NPR))
    b1 = jnp.tile(brel1.reshape(1, H), (1, NPR))
    b2 = jnp.tile(brel2.reshape(1, H), (1, NPR))

    sc = _get_sc_seg_sum()

    z0, r0 = _tc0(x0_pk, w0r, w0o)
    s0 = sc(z0.reshape(NC * N, HH), src2, src2p1, dst).reshape(PN, 128)
    z1, r1 = _tc_layer(s0, r0, b0, w1r, w1o)
    s1 = sc(z1.reshape(NC * N, HH), src2, src2p1, dst).reshape(PN, 128)
    z2, r2 = _tc_layer(s1, r1, b1, w2r, w2o)
    s2 = sc(z2.reshape(NC * N, HH), src2, src2p1, dst).reshape(PN, 128)
    sums, cnts = _tc_pool(s2, r2, b2, batch_tp)
    out = _tc_mlp(sums, cnts,
                  W5, b5.reshape(1, H), W6, b6.reshape(1, H),
                  W7, b7.reshape(1, H), W8, b8.reshape(1, H),
                  Wf, bf.reshape(1, OUT))
    return out
